# baseline (speedup 1.0000x reference)
import dataclasses
import functools

import jax
import jax.numpy as jnp
import numpy as np
from jax import lax
from jax.experimental import pallas as pl
from jax.experimental.pallas import tpu as pltpu
from jax.experimental.pallas import tpu_sc as plsc

BATCH = 1024
VOCAB = 1000
T_STEPS = 200
TCH = 40
NCH = T_STEPS // TCH
D_DIM = 128
DW = D_DIM // 2
DECAY = 0.9

NUM_CORES = 2
NUM_SUBCORES = 16
NUM_WORKERS = NUM_CORES * NUM_SUBCORES
ROWS_PER_WORKER = BATCH // NUM_WORKERS
LANES = 16
DC = D_DIM // LANES
NGRP = D_DIM // 32
NFB = 4
NEB = 2

_PERM = np.arange(D_DIM).reshape(NGRP, 2, LANES).transpose(0, 2, 1).reshape(-1)


def kernel(ctrl_tokens, prev_trace, embed_table):
    idx = ctrl_tokens[:, :, 1].astype(jnp.int32).reshape(BATCH * T_STEPS)
    table_bf = embed_table.astype(jnp.bfloat16)[:, _PERM]
    table_i32 = lax.bitcast_convert_type(
        table_bf.reshape(VOCAB, DW, 2), jnp.int32)

    mesh = plsc.VectorSubcoreMesh(core_axis_name="c", subcore_axis_name="s")

    cparams = pltpu.CompilerParams()
    if "needs_layout_passes" in pltpu.CompilerParams.__dataclass_fields__:
        cparams = dataclasses.replace(cparams, needs_layout_passes=False)

    @functools.partial(
        pl.kernel,
        out_type=jax.ShapeDtypeStruct((BATCH * T_STEPS, D_DIM), jnp.float32),
        mesh=mesh,
        compiler_params=cparams,
        scratch_types=[
            pltpu.VMEM((NFB, TCH, D_DIM), jnp.float32),
            pltpu.VMEM((NEB, T_STEPS, DW), jnp.int32),
            pltpu.VMEM((4 * T_STEPS,), jnp.int32),
            pltpu.VMEM((4, D_DIM), jnp.float32),
            pltpu.SemaphoreType.DMA((NEB,)),
            pltpu.SemaphoreType.DMA((NFB,)),
            pltpu.SemaphoreType.DMA((4,)),
            pltpu.SemaphoreType.DMA((4,)),
            pltpu.VMEM_SHARED((VOCAB, DW), jnp.int32),
        ],
    )
    def ev_kernel(idx_hbm, prev_hbm, table_hbm, out_hbm,
                  fbuf, ebuf, ibuf, pbuf, gsem, osem, isem, psem, table_sh):
        wid = lax.axis_index("s") * NUM_CORES + lax.axis_index("c")
        base = wid * ROWS_PER_WORKER
        @pl.when(lax.axis_index("s") == 0)
        def _():
            pltpu.sync_copy(table_hbm, table_sh)
        plsc.subcore_barrier()

        def idx_copy(r, b):
            return pltpu.make_async_copy(
                idx_hbm.at[pl.ds((base + r) * T_STEPS, T_STEPS)],
                ibuf.at[pl.ds(b * T_STEPS, T_STEPS)], isem.at[b])

        def prev_copy(r, b):
            return pltpu.make_async_copy(
                prev_hbm.at[base + r], pbuf.at[b], psem.at[b])

        def gather(r, e, b):
            return pltpu.make_async_copy(
                table_sh.at[ibuf.at[pl.ds(b * T_STEPS, T_STEPS)]],
                ebuf.at[e], gsem.at[e])

        def out_copy(r, c, b):
            return pltpu.make_async_copy(
                fbuf.at[b],
                out_hbm.at[pl.ds((base + r) * T_STEPS + c * TCH, TCH)],
                osem.at[b])

        for b in range(4):
            idx_copy(b, b).start()
            prev_copy(b, b).start()
        for e in range(NEB):
            idx_copy(e, e).wait()
            gather(e, e, e).start()

        @pl.loop(0, ROWS_PER_WORKER, step=4)
        def _(rbase):
            for j in range(4):
                e = j % NEB
                r = rbase + j
                gather(r, e, j).wait()

                @pl.when(r < ROWS_PER_WORKER - 4)
                def _():
                    idx_copy(r + 4, j).start()
                    prev_copy(r + 4, j).start()
                prev_copy(r, j).wait()

                acc = tuple(pbuf[j, pl.ds(16 * k, 16)] for k in range(DC))
                for c in range(NCH):
                    fb = (j + c) % NFB

                    if c >= 4:
                        out_copy(r, c, fb).wait()
                    else:
                        @pl.when(r >= 1)
                        def _():
                            out_copy(r, c, fb).wait()

                    def step(th, acc, c=c, fb=fb):
                        acc = list(acc)
                        t = c * TCH + th
                        for g in range(NGRP):
                            words = ebuf[e, t, pl.ds(16 * g, 16)]
                            pair = plsc.bitcast(words, jnp.bfloat16)
                            lo, hi = plsc.unpack(
                                pair, format=plsc.PackFormat.INTERLEAVED)
                            acc[2 * g] = lo + DECAY * acc[2 * g]
                            acc[2 * g + 1] = hi + DECAY * acc[2 * g + 1]
                            fbuf[fb, th, pl.ds(32 * g, 16)] = acc[2 * g]
                            fbuf[fb, th, pl.ds(32 * g + 16, 16)] = acc[2 * g + 1]
                        return tuple(acc)

                    acc = lax.fori_loop(0, TCH, step, acc)
                    out_copy(r, c, fb).start()

                @pl.when(r < ROWS_PER_WORKER - 2)
                def _():
                    idx_copy(r + 2, (j + 2) % 4).wait()
                    gather(r + 2, e, (j + 2) % 4).start()

        for c in range(1, NCH):
            out_copy(ROWS_PER_WORKER - 1, c, (3 + c) % NFB).wait()

    out = ev_kernel(idx, prev_trace, table_i32)
    return out.reshape(BATCH, T_STEPS, D_DIM)
